# baseline (speedup 1.0000x reference)
import jax
import jax.numpy as jnp
from jax import lax
from jax.experimental import pallas as pl
from jax.experimental.pallas import tpu as pltpu
from jax.experimental.pallas import tpu_sc as plsc

B = 16384
D = 64
NW = 32
BPW = B // NW
GC = 128
NCH = BPW // GC
L = 16


def _body(h_hbm, r_hbm, t_hbm, ent_hbm, rel_hbm, out_hbm,
          hi_v, ri_v, ti_v, hrow, rrow, trow, out_v, sem):
    cid = lax.axis_index("c")
    sid = lax.axis_index("s")
    wid = sid * 2 + cid
    base = wid * BPW

    pltpu.sync_copy(h_hbm.at[pl.ds(base, BPW)], hi_v)
    pltpu.sync_copy(r_hbm.at[pl.ds(base, BPW)], ri_v)
    pltpu.sync_copy(t_hbm.at[pl.ds(base, BPW)], ti_v)

    copies = []
    for k in range(NCH):
        off = k * GC
        copies.append(pltpu.async_copy(
            ent_hbm.at[hi_v.at[pl.ds(off, GC)]],
            hrow.at[pl.ds(off, GC), :], sem))
        copies.append(pltpu.async_copy(
            ent_hbm.at[ti_v.at[pl.ds(off, GC)]],
            trow.at[pl.ds(off, GC), :], sem))
        copies.append(pltpu.async_copy(
            rel_hbm.at[ri_v.at[pl.ds(off, GC)]],
            rrow.at[pl.ds(off, GC), :], sem))
    for cp in copies:
        cp.wait()

    iota = lax.iota(jnp.int32, L)

    def group(g, _):
        iv = g * L + iota
        acc = jnp.zeros((L,), jnp.float32)
        for j in range(D):
            jv = jnp.full((L,), j, jnp.int32)
            hj = plsc.load_gather(hrow, [iv, jv])
            rj = plsc.load_gather(rrow, [iv, jv])
            tj = plsc.load_gather(trow, [iv, jv])
            acc = acc + jnp.abs(hj + rj - tj)
        out_v[pl.ds(g * L, L)] = -acc
        return 0

    lax.fori_loop(0, BPW // L, group, 0)

    pltpu.sync_copy(out_v, out_hbm.at[pl.ds(base, BPW)])


@jax.jit
def kernel(h, r, t, entity_table, relation_table):
    k = pl.kernel(
        _body,
        mesh=plsc.VectorSubcoreMesh(core_axis_name="c", subcore_axis_name="s"),
        out_type=jax.ShapeDtypeStruct((B,), jnp.float32),
        compiler_params=pltpu.CompilerParams(
            needs_layout_passes=False, use_tc_tiling_on_sc=False),
        scratch_types=[
            pltpu.VMEM((BPW,), jnp.int32),
            pltpu.VMEM((BPW,), jnp.int32),
            pltpu.VMEM((BPW,), jnp.int32),
            pltpu.VMEM((BPW, D), jnp.float32),
            pltpu.VMEM((BPW, D), jnp.float32),
            pltpu.VMEM((BPW, D), jnp.float32),
            pltpu.VMEM((BPW,), jnp.float32),
            pltpu.SemaphoreType.DMA,
        ],
    )
    return k(h, r, t, entity_table, relation_table)


# speedup vs baseline: 1.0095x; 1.0095x over previous
import jax
import jax.numpy as jnp
from jax import lax
from jax.experimental import pallas as pl
from jax.experimental.pallas import tpu as pltpu
from jax.experimental.pallas import tpu_sc as plsc

B = 16384
D = 64
NW = 32
BPW = B // NW
PCH = 128
NP = BPW // PCH
L = 16
NG = PCH // L


def _body(h_hbm, r_hbm, t_hbm, ent_hbm, rel_hbm, out_hbm,
          hi, ti, ri, hs, ts, rs, hoff, toff, roff,
          hb0, tb0, rb0, hb1, tb1, rb1, out_v, sem0, sem1):
    cid = lax.axis_index("c")
    sid = lax.axis_index("s")
    wid = sid * 2 + cid
    base = wid * BPW

    pltpu.sync_copy(h_hbm.at[pl.ds(base, BPW)], hi)
    pltpu.sync_copy(t_hbm.at[pl.ds(base, BPW)], ti)
    pltpu.sync_copy(r_hbm.at[pl.ds(base, BPW)], ri)

    def split(g, _):
        s = pl.ds(g * L, L)
        for src, rows, offs in ((hi, hs, hoff), (ti, ts, toff), (ri, rs, roff)):
            e = src[s]
            rows[s] = lax.shift_right_logical(e, 1)
            offs[s] = (e & 1) * D
        return 0

    lax.fori_loop(0, BPW // L, split, 0)

    hb = (hb0, hb1)
    tb = (tb0, tb1)
    rb = (rb0, rb1)
    sems = (sem0, sem1)

    def fire(p, slot):
        s = pl.ds(p * PCH, PCH)
        return (
            pltpu.async_copy(ent_hbm.at[hs.at[s]], hb[slot], sems[slot]),
            pltpu.async_copy(ent_hbm.at[ts.at[s]], tb[slot], sems[slot]),
            pltpu.async_copy(rel_hbm.at[rs.at[s]], rb[slot], sems[slot]),
        )

    lanes = lax.broadcasted_iota(jnp.int32, (L,), 0)

    def score(p, slot):
        def group(g, _):
            gbase = g * L
            ivec = lanes + gbase
            s = pl.ds(p * PCH + gbase, L)
            ho = hoff[s]
            to = toff[s]
            ro = roff[s]
            acc = jnp.zeros((L,), jnp.float32)
            for j in range(D):
                hj = plsc.load_gather(hb[slot], [ivec, ho + j])
                tj = plsc.load_gather(tb[slot], [ivec, to + j])
                rj = plsc.load_gather(rb[slot], [ivec, ro + j])
                acc = acc + jnp.abs(hj + rj - tj)
            out_v[pl.ds(p * PCH + gbase, L)] = -acc
            return 0

        lax.fori_loop(0, NG, group, 0)

    cps = fire(0, 0)
    for p in range(NP):
        slot = p % 2
        if p + 1 < NP:
            nxt = fire(p + 1, 1 - slot)
        for cp in cps:
            cp.wait()
        score(p, slot)
        if p + 1 < NP:
            cps = nxt

    pltpu.sync_copy(out_v, out_hbm.at[pl.ds(base, BPW)])


@jax.jit
def kernel(h, r, t, entity_table, relation_table):
    ne, d = entity_table.shape
    nr, _ = relation_table.shape
    ent2 = entity_table.reshape(ne // 2, 2 * d)
    rel2 = relation_table.reshape(nr // 2, 2 * d)
    k = pl.kernel(
        _body,
        mesh=plsc.VectorSubcoreMesh(core_axis_name="c", subcore_axis_name="s"),
        out_type=jax.ShapeDtypeStruct((B,), jnp.float32),
        compiler_params=pltpu.CompilerParams(
            needs_layout_passes=False,
        ),
        scratch_types=[
            pltpu.VMEM((BPW,), jnp.int32),
            pltpu.VMEM((BPW,), jnp.int32),
            pltpu.VMEM((BPW,), jnp.int32),
            pltpu.VMEM((BPW,), jnp.int32),
            pltpu.VMEM((BPW,), jnp.int32),
            pltpu.VMEM((BPW,), jnp.int32),
            pltpu.VMEM((BPW,), jnp.int32),
            pltpu.VMEM((BPW,), jnp.int32),
            pltpu.VMEM((BPW,), jnp.int32),
            pltpu.VMEM((PCH, 2 * D), jnp.float32),
            pltpu.VMEM((PCH, 2 * D), jnp.float32),
            pltpu.VMEM((PCH, 2 * D), jnp.float32),
            pltpu.VMEM((PCH, 2 * D), jnp.float32),
            pltpu.VMEM((PCH, 2 * D), jnp.float32),
            pltpu.VMEM((PCH, 2 * D), jnp.float32),
            pltpu.VMEM((BPW,), jnp.float32),
            pltpu.SemaphoreType.DMA,
            pltpu.SemaphoreType.DMA,
        ],
    )
    return k(h, r, t, ent2, rel2)


# speedup vs baseline: 1.0743x; 1.0642x over previous
import jax
import jax.numpy as jnp
from jax import lax
from jax.experimental import pallas as pl
from jax.experimental.pallas import tpu as pltpu
from jax.experimental.pallas import tpu_sc as plsc

B = 16384
D = 64
NW = 32
BPW = B // NW
PCH = 128
NP = BPW // PCH
L = 16
NC = D // L


def _body(h_hbm, r_hbm, t_hbm, ent_hbm, rel_hbm, out_hbm,
          hi, ti, ri,
          hb0, tb0, rb0, hb1, tb1, rb1, out_v, sem0, sem1):
    cid = lax.axis_index("c")
    sid = lax.axis_index("s")
    wid = sid * 2 + cid
    base = wid * BPW

    pltpu.sync_copy(h_hbm.at[pl.ds(base, BPW)], hi)
    pltpu.sync_copy(t_hbm.at[pl.ds(base, BPW)], ti)
    pltpu.sync_copy(r_hbm.at[pl.ds(base, BPW)], ri)

    hb = (hb0, hb1)
    tb = (tb0, tb1)
    rb = (rb0, rb1)
    sems = (sem0, sem1)

    def fire(p, slot):
        s = pl.ds(p * PCH, PCH)
        return (
            pltpu.async_copy(ent_hbm.at[hi.at[s]], hb[slot], sems[slot]),
            pltpu.async_copy(ent_hbm.at[ti.at[s]], tb[slot], sems[slot]),
            pltpu.async_copy(rel_hbm.at[ri.at[s]], rb[slot], sems[slot]),
        )

    lanes = lax.broadcasted_iota(jnp.int32, (L,), 0)
    last = lanes == (L - 1)
    zeros_i = jnp.zeros((L,), jnp.int32)

    def score(p, slot):
        hs, ts, rs = hb[slot], tb[slot], rb[slot]

        def item(i, _):
            acc = jnp.zeros((L,), jnp.float32)
            for k in range(NC):
                s = pl.ds(k * L, L)
                hv = hs[i, s]
                tv = ts[i, s]
                rv = rs[i, s]
                acc = acc + jnp.abs(hv + rv - tv)
            cs = plsc.cumsum(acc)
            iv = zeros_i + (p * PCH + i)
            plsc.store_scatter(out_v, [iv], -cs, mask=last)
            return 0

        lax.fori_loop(0, PCH, item, 0)

    cps = fire(0, 0)
    for p in range(NP):
        slot = p % 2
        if p + 1 < NP:
            nxt = fire(p + 1, 1 - slot)
        for cp in cps:
            cp.wait()
        score(p, slot)
        if p + 1 < NP:
            cps = nxt

    pltpu.sync_copy(out_v, out_hbm.at[pl.ds(base, BPW)])


@jax.jit
def kernel(h, r, t, entity_table, relation_table):
    k = pl.kernel(
        _body,
        mesh=plsc.VectorSubcoreMesh(core_axis_name="c", subcore_axis_name="s"),
        out_type=jax.ShapeDtypeStruct((B,), jnp.float32),
        compiler_params=pltpu.CompilerParams(
            needs_layout_passes=False,
            use_tc_tiling_on_sc=False,
        ),
        scratch_types=[
            pltpu.VMEM((BPW,), jnp.int32),
            pltpu.VMEM((BPW,), jnp.int32),
            pltpu.VMEM((BPW,), jnp.int32),
            pltpu.VMEM((PCH, D), jnp.float32),
            pltpu.VMEM((PCH, D), jnp.float32),
            pltpu.VMEM((PCH, D), jnp.float32),
            pltpu.VMEM((PCH, D), jnp.float32),
            pltpu.VMEM((PCH, D), jnp.float32),
            pltpu.VMEM((PCH, D), jnp.float32),
            pltpu.VMEM((BPW,), jnp.float32),
            pltpu.SemaphoreType.DMA,
            pltpu.SemaphoreType.DMA,
        ],
    )
    return k(h, r, t, entity_table, relation_table)
